# baseline (speedup 1.0000x reference)
import functools

import jax
import jax.numpy as jnp
from jax import lax
from jax.experimental import pallas as pl
from jax.experimental.pallas import tpu as pltpu
from jax.experimental.pallas import tpu_sc as plsc

_NUM_WORKERS = 32
_CHUNK = 64
_RING = 4


def kernel(x, pe_table):
    b, size, dx = x.shape
    dim = pe_table.shape[-1]
    rows = size // _NUM_WORKERS
    nchunks = rows // _CHUNK
    total = b * nchunks
    mesh = plsc.VectorSubcoreMesh(core_axis_name="c", subcore_axis_name="s")

    @functools.partial(
        pl.kernel,
        mesh=mesh,
        out_type=jax.ShapeDtypeStruct((b, size, dx + dim), x.dtype),
        scratch_types=[
            pltpu.MemorySpace.VMEM((rows, dim), x.dtype),
            pltpu.MemorySpace.VMEM((_RING, _CHUNK, dx), x.dtype),
            pltpu.SemaphoreType.DMA,
            pltpu.SemaphoreType.DMA((_RING,)),
            pltpu.SemaphoreType.DMA((_RING,)),
            pltpu.SemaphoreType.DMA,
        ],
    )
    def run(x_hbm, pe_hbm, out_hbm, pebuf, xbuf, sem_pe, sem_xl, sem_xs, sem_ps):
        wid = lax.axis_index("s") * 2 + lax.axis_index("c")
        s0 = wid * rows
        pe_load = pltpu.make_async_copy(pe_hbm.at[pl.ds(s0, rows), :], pebuf, sem_pe)
        pe_load.start()

        def chunk_coords(i):
            return i // nchunks, (i % nchunks) * _CHUNK

        x_loads = []
        x_stores = []
        for i in range(total):
            bb, r0 = chunk_coords(i)
            slot = i % _RING
            x_loads.append(
                pltpu.make_async_copy(
                    x_hbm.at[bb, pl.ds(s0 + r0, _CHUNK), :],
                    xbuf.at[slot],
                    sem_xl.at[slot],
                )
            )
            x_stores.append(
                pltpu.make_async_copy(
                    xbuf.at[slot],
                    out_hbm.at[bb, pl.ds(s0 + r0, _CHUNK), pl.ds(0, dx)],
                    sem_xs.at[slot],
                )
            )
        pe_stores = [
            pltpu.make_async_copy(
                pebuf, out_hbm.at[bb, pl.ds(s0, rows), pl.ds(dx, dim)], sem_ps
            )
            for bb in range(b)
        ]

        for i in range(min(_RING, total)):
            x_loads[i].start()
        pe_load.wait()
        for st in pe_stores:
            st.start()
        for i in range(total):
            x_loads[i].wait()
            x_stores[i].start()
            if i + _RING < total:
                x_stores[i].wait()
                x_loads[i + _RING].start()
        for i in range(max(0, total - _RING), total):
            x_stores[i].wait()
        for st in pe_stores:
            st.wait()

    return run(x, pe_table)
